# Initial kernel scaffold; baseline (speedup 1.0000x reference)
import jax
import jax.numpy as jnp
from jax.experimental import pallas as pl


def kernel(x, emb, W1, b1, W2, b2, W3, b3):
    raise NotImplementedError("write your pallas kernel here")


# speedup vs baseline: 12.1153x; 12.1153x over previous
import functools

import jax
import jax.numpy as jnp
from jax import lax
from jax.experimental import pallas as pl
from jax.experimental.pallas import tpu as pltpu
from jax.experimental.pallas import tpu_sc as plsc

NC = 2
NS = 16
NW = NC * NS


def _sc_gather(xf, emb, chunk):
  n = xf.shape[0]
  d = emb.shape[1]
  idx_w = n // NW
  nchunk = idx_w // chunk

  mesh = plsc.VectorSubcoreMesh(core_axis_name="c", subcore_axis_name="s")

  @functools.partial(
      pl.kernel,
      mesh=mesh,
      out_type=jax.ShapeDtypeStruct((n, d), jnp.float32),
      compiler_params=pltpu.CompilerParams(use_tc_tiling_on_sc=False),
      scratch_types=[
          pltpu.VMEM((chunk,), jnp.int32),
          pltpu.VMEM((chunk, d), jnp.float32),
          pltpu.SemaphoreType.DMA,
      ],
  )
  def k(x_hbm, emb_hbm, out_hbm, idx_v, rows_v, sem):
    wid = lax.axis_index("s") * NC + lax.axis_index("c")
    base = wid * idx_w

    def step(i, carry):
      off = base + i * chunk
      pltpu.sync_copy(x_hbm.at[pl.ds(off, chunk)], idx_v)
      pltpu.async_copy(emb_hbm.at[idx_v], rows_v, sem).wait()
      pltpu.sync_copy(rows_v, out_hbm.at[pl.ds(off, chunk)])
      return carry

    lax.fori_loop(0, nchunk, step, 0)

  return k(xf, emb)


def _mlp_body(e_ref, w1_ref, b1_ref, w2_ref, b2_ref, w3_ref, b3_ref, o_ref):
  e = e_ref[...]
  h = jnp.dot(e, w1_ref[...], preferred_element_type=jnp.float32)
  h = jnp.maximum(h + b1_ref[...], 0.0)
  h = jnp.dot(h, w2_ref[...], preferred_element_type=jnp.float32)
  h = jnp.maximum(h + b2_ref[...], 0.0)
  o = jnp.dot(h, w3_ref[...], preferred_element_type=jnp.float32)
  o_ref[...] = o + b3_ref[...]


def _tc_mlp(e, W1, b1, W2, b2, W3, b3, tb):
  bsz, f = e.shape
  h1 = W1.shape[1]
  h2 = W2.shape[1]
  ncls = W3.shape[1]
  grid = (bsz // tb,)
  return pl.pallas_call(
      _mlp_body,
      grid=grid,
      in_specs=[
          pl.BlockSpec((tb, f), lambda i: (i, 0)),
          pl.BlockSpec((f, h1), lambda i: (0, 0)),
          pl.BlockSpec((1, h1), lambda i: (0, 0)),
          pl.BlockSpec((h1, h2), lambda i: (0, 0)),
          pl.BlockSpec((1, h2), lambda i: (0, 0)),
          pl.BlockSpec((h2, ncls), lambda i: (0, 0)),
          pl.BlockSpec((1, ncls), lambda i: (0, 0)),
      ],
      out_specs=pl.BlockSpec((tb, ncls), lambda i: (i, 0)),
      out_shape=jax.ShapeDtypeStruct((bsz, ncls), jnp.float32),
  )(e, W1, b1, W2, b2, W3, b3)


@jax.jit
def kernel(x, emb, W1, b1, W2, b2, W3, b3):
  b, s = x.shape
  v, d = emb.shape
  xf = x.reshape(-1).astype(jnp.int32)
  e = _sc_gather(xf, emb, chunk=2048)
  e2 = e.reshape(b, s * d)
  return _tc_mlp(e2, W1, b1.reshape(1, -1), W2, b2.reshape(1, -1),
                 W3, b3.reshape(1, -1), tb=512)


# speedup vs baseline: 12.1712x; 1.0046x over previous
import functools

import jax
import jax.numpy as jnp
from jax import lax
from jax.experimental import pallas as pl
from jax.experimental.pallas import tpu as pltpu
from jax.experimental.pallas import tpu_sc as plsc

NC = 2
NS = 16
NW = NC * NS


def _sc_gather(xf, emb, chunk):
  n = xf.shape[0]
  d = emb.shape[1]
  idx_w = n // NW
  nchunk = idx_w // chunk

  mesh = plsc.VectorSubcoreMesh(core_axis_name="c", subcore_axis_name="s")

  nh = nchunk // 2

  @functools.partial(
      pl.kernel,
      mesh=mesh,
      out_type=jax.ShapeDtypeStruct((n, d), jnp.float32),
      compiler_params=pltpu.CompilerParams(use_tc_tiling_on_sc=False),
      scratch_types=[
          pltpu.VMEM((chunk,), jnp.int32),
          pltpu.VMEM((chunk,), jnp.int32),
          pltpu.VMEM((chunk, d), jnp.float32),
          pltpu.VMEM((chunk, d), jnp.float32),
          pltpu.SemaphoreType.DMA,
          pltpu.SemaphoreType.DMA,
          pltpu.SemaphoreType.DMA,
          pltpu.SemaphoreType.DMA,
          pltpu.SemaphoreType.DMA,
          pltpu.SemaphoreType.DMA,
      ],
  )
  def k(x_hbm, emb_hbm, out_hbm, idx0, idx1, rows0, rows1,
        isem0, isem1, gsem0, gsem1, osem0, osem1):
    wid = lax.axis_index("s") * NC + lax.axis_index("c")
    base = wid * idx_w

    def xs(i):
      return x_hbm.at[pl.ds(base + i * chunk, chunk)]

    def os(i):
      return out_hbm.at[pl.ds(base + i * chunk, chunk)]

    pltpu.async_copy(xs(0), idx0, isem0)
    pltpu.async_copy(xs(1), idx1, isem1)

    def step(j, carry):
      a = 2 * j
      b = a + 1
      pltpu.make_async_copy(xs(a), idx0, isem0).wait()

      @pl.when(j > 0)
      def _():
        pltpu.make_async_copy(rows0, os(a), osem0).wait()

      pltpu.async_copy(emb_hbm.at[idx0], rows0, gsem0)

      pltpu.make_async_copy(xs(b), idx1, isem1).wait()

      @pl.when(j > 0)
      def _():
        pltpu.make_async_copy(rows1, os(b), osem1).wait()

      pltpu.async_copy(emb_hbm.at[idx1], rows1, gsem1)

      pltpu.make_async_copy(emb_hbm.at[idx0], rows0, gsem0).wait()
      pltpu.async_copy(rows0, os(a), osem0)

      @pl.when(j < nh - 1)
      def _():
        pltpu.async_copy(xs(a + 2), idx0, isem0)

      pltpu.make_async_copy(emb_hbm.at[idx1], rows1, gsem1).wait()
      pltpu.async_copy(rows1, os(b), osem1)

      @pl.when(j < nh - 1)
      def _():
        pltpu.async_copy(xs(b + 2), idx1, isem1)

      return carry

    lax.fori_loop(0, nh, step, 0)
    pltpu.make_async_copy(rows0, os(nchunk - 2), osem0).wait()
    pltpu.make_async_copy(rows1, os(nchunk - 1), osem1).wait()

  return k(xf, emb)


def _mlp_body(e_ref, w1_ref, b1_ref, w2_ref, b2_ref, w3_ref, b3_ref, o_ref):
  e = e_ref[...]
  h = jnp.dot(e, w1_ref[...], preferred_element_type=jnp.float32)
  h = jnp.maximum(h + b1_ref[...], 0.0)
  h = jnp.dot(h, w2_ref[...], preferred_element_type=jnp.float32)
  h = jnp.maximum(h + b2_ref[...], 0.0)
  o = jnp.dot(h, w3_ref[...], preferred_element_type=jnp.float32)
  o_ref[...] = o + b3_ref[...]


def _tc_mlp(e, W1, b1, W2, b2, W3, b3, tb):
  bsz, f = e.shape
  h1 = W1.shape[1]
  h2 = W2.shape[1]
  ncls = W3.shape[1]
  grid = (bsz // tb,)
  return pl.pallas_call(
      _mlp_body,
      grid=grid,
      in_specs=[
          pl.BlockSpec((tb, f), lambda i: (i, 0)),
          pl.BlockSpec((f, h1), lambda i: (0, 0)),
          pl.BlockSpec((1, h1), lambda i: (0, 0)),
          pl.BlockSpec((h1, h2), lambda i: (0, 0)),
          pl.BlockSpec((1, h2), lambda i: (0, 0)),
          pl.BlockSpec((h2, ncls), lambda i: (0, 0)),
          pl.BlockSpec((1, ncls), lambda i: (0, 0)),
      ],
      out_specs=pl.BlockSpec((tb, ncls), lambda i: (i, 0)),
      out_shape=jax.ShapeDtypeStruct((bsz, ncls), jnp.float32),
  )(e, W1, b1, W2, b2, W3, b3)


@jax.jit
def kernel(x, emb, W1, b1, W2, b2, W3, b3):
  b, s = x.shape
  v, d = emb.shape
  xf = x.reshape(-1).astype(jnp.int32)
  e = _sc_gather(xf, emb, chunk=2048)
  e2 = e.reshape(b, s * d)
  return _tc_mlp(e2, W1, b1.reshape(1, -1), W2, b2.reshape(1, -1),
                 W3, b3.reshape(1, -1), tb=512)


# speedup vs baseline: 27.5243x; 2.2614x over previous
import functools

import jax
import jax.numpy as jnp
from jax import lax
from jax.experimental import pallas as pl
from jax.experimental.pallas import tpu as pltpu
from jax.experimental.pallas import tpu_sc as plsc

NC = 2
NS = 16
NW = NC * NS


def _sc_gather(xf, emb, chunk):
  n = xf.shape[0]
  d = emb.shape[1]
  idx_w = n // NW
  nchunk = idx_w // chunk

  mesh = plsc.VectorSubcoreMesh(core_axis_name="c", subcore_axis_name="s")

  nh = nchunk // 2

  @functools.partial(
      pl.kernel,
      mesh=mesh,
      out_type=jax.ShapeDtypeStruct((n, d), jnp.float32),
      compiler_params=pltpu.CompilerParams(use_tc_tiling_on_sc=False),
      scratch_types=[
          pltpu.VMEM((chunk,), jnp.int32),
          pltpu.VMEM((chunk,), jnp.int32),
          pltpu.VMEM((chunk, d), jnp.float32),
          pltpu.VMEM((chunk, d), jnp.float32),
          pltpu.VMEM_SHARED(emb.shape, jnp.float32),
          pltpu.SemaphoreType.DMA,
          pltpu.SemaphoreType.DMA,
          pltpu.SemaphoreType.DMA,
          pltpu.SemaphoreType.DMA,
          pltpu.SemaphoreType.DMA,
          pltpu.SemaphoreType.DMA,
      ],
  )
  def k(x_hbm, emb_hbm, out_hbm, idx0, idx1, rows0, rows1, emb_v,
        isem0, isem1, gsem0, gsem1, osem0, osem1):
    wid = lax.axis_index("s") * NC + lax.axis_index("c")
    base = wid * idx_w
    @pl.when(lax.axis_index("s") == 0)
    def _():
      pltpu.sync_copy(emb_hbm, emb_v)

    plsc.subcore_barrier()

    def xs(i):
      return x_hbm.at[pl.ds(base + i * chunk, chunk)]

    def os(i):
      return out_hbm.at[pl.ds(base + i * chunk, chunk)]

    pltpu.async_copy(xs(0), idx0, isem0)
    pltpu.async_copy(xs(1), idx1, isem1)

    def step(j, carry):
      a = 2 * j
      b = a + 1
      pltpu.make_async_copy(xs(a), idx0, isem0).wait()

      @pl.when(j > 0)
      def _():
        pltpu.make_async_copy(rows0, os(a), osem0).wait()

      pltpu.async_copy(emb_v.at[idx0], rows0, gsem0)

      pltpu.make_async_copy(xs(b), idx1, isem1).wait()

      @pl.when(j > 0)
      def _():
        pltpu.make_async_copy(rows1, os(b), osem1).wait()

      pltpu.async_copy(emb_v.at[idx1], rows1, gsem1)

      pltpu.make_async_copy(emb_v.at[idx0], rows0, gsem0).wait()
      pltpu.async_copy(rows0, os(a), osem0)

      @pl.when(j < nh - 1)
      def _():
        pltpu.async_copy(xs(a + 2), idx0, isem0)

      pltpu.make_async_copy(emb_v.at[idx1], rows1, gsem1).wait()
      pltpu.async_copy(rows1, os(b), osem1)

      @pl.when(j < nh - 1)
      def _():
        pltpu.async_copy(xs(b + 2), idx1, isem1)

      return carry

    lax.fori_loop(0, nh, step, 0)
    pltpu.make_async_copy(rows0, os(nchunk - 2), osem0).wait()
    pltpu.make_async_copy(rows1, os(nchunk - 1), osem1).wait()

  return k(xf, emb)


def _mlp_body(e_ref, w1_ref, b1_ref, w2_ref, b2_ref, w3_ref, b3_ref, o_ref):
  e = e_ref[...]
  h = jnp.dot(e, w1_ref[...], preferred_element_type=jnp.float32)
  h = jnp.maximum(h + b1_ref[...], 0.0)
  h = jnp.dot(h, w2_ref[...], preferred_element_type=jnp.float32)
  h = jnp.maximum(h + b2_ref[...], 0.0)
  o = jnp.dot(h, w3_ref[...], preferred_element_type=jnp.float32)
  o_ref[...] = o + b3_ref[...]


def _tc_mlp(e, W1, b1, W2, b2, W3, b3, tb):
  bsz, f = e.shape
  h1 = W1.shape[1]
  h2 = W2.shape[1]
  ncls = W3.shape[1]
  grid = (bsz // tb,)
  return pl.pallas_call(
      _mlp_body,
      grid=grid,
      in_specs=[
          pl.BlockSpec((tb, f), lambda i: (i, 0)),
          pl.BlockSpec((f, h1), lambda i: (0, 0)),
          pl.BlockSpec((1, h1), lambda i: (0, 0)),
          pl.BlockSpec((h1, h2), lambda i: (0, 0)),
          pl.BlockSpec((1, h2), lambda i: (0, 0)),
          pl.BlockSpec((h2, ncls), lambda i: (0, 0)),
          pl.BlockSpec((1, ncls), lambda i: (0, 0)),
      ],
      out_specs=pl.BlockSpec((tb, ncls), lambda i: (i, 0)),
      out_shape=jax.ShapeDtypeStruct((bsz, ncls), jnp.float32),
  )(e, W1, b1, W2, b2, W3, b3)


@jax.jit
def kernel(x, emb, W1, b1, W2, b2, W3, b3):
  b, s = x.shape
  v, d = emb.shape
  xf = x.reshape(-1).astype(jnp.int32)
  e = _sc_gather(xf, emb, chunk=2048)
  e2 = e.reshape(b, s * d)
  return _tc_mlp(e2, W1, b1.reshape(1, -1), W2, b2.reshape(1, -1),
                 W3, b3.reshape(1, -1), tb=512)


# speedup vs baseline: 33.2411x; 1.2077x over previous
import functools

import jax
import jax.numpy as jnp
from jax import lax
from jax.experimental import pallas as pl
from jax.experimental.pallas import tpu as pltpu
from jax.experimental.pallas import tpu_sc as plsc

NC = 2
NS = 16
NW = NC * NS


def _sc_gather(xf, emb, chunk):
  n = xf.shape[0]
  d = emb.shape[1]
  idx_w = n // NW
  nchunk = idx_w // chunk
  nh = nchunk // 2

  mesh = plsc.VectorSubcoreMesh(core_axis_name="c", subcore_axis_name="s")

  @functools.partial(
      pl.kernel,
      mesh=mesh,
      out_type=jax.ShapeDtypeStruct((n, d), jnp.float32),
      compiler_params=pltpu.CompilerParams(use_tc_tiling_on_sc=False),
      scratch_types=[
          pltpu.VMEM((chunk,), jnp.int32),
          pltpu.VMEM((chunk,), jnp.int32),
          pltpu.VMEM((chunk, d), jnp.float32),
          pltpu.VMEM((chunk, d), jnp.float32),
          pltpu.VMEM_SHARED(emb.shape, jnp.float32),
          pltpu.SemaphoreType.DMA,
          pltpu.SemaphoreType.DMA,
          pltpu.SemaphoreType.DMA,
          pltpu.SemaphoreType.DMA,
          pltpu.SemaphoreType.DMA,
          pltpu.SemaphoreType.DMA,
      ],
  )
  def k(x_hbm, emb_hbm, out_hbm, idx0, idx1, rows0, rows1, emb_v,
        isem0, isem1, gsem0, gsem1, osem0, osem1):
    wid = lax.axis_index("s") * NC + lax.axis_index("c")
    base = wid * idx_w
    @pl.when(lax.axis_index("s") == 0)
    def _():
      pltpu.sync_copy(emb_hbm, emb_v)

    plsc.subcore_barrier()

    def xs(i):
      return x_hbm.at[pl.ds(base + i * chunk, chunk)]

    def os(i):
      return out_hbm.at[pl.ds(base + i * chunk, chunk)]

    pltpu.async_copy(xs(0), idx0, isem0)
    pltpu.async_copy(xs(1), idx1, isem1)

    def step(j, carry):
      a = 2 * j
      b = a + 1
      pltpu.make_async_copy(xs(a), idx0, isem0).wait()

      @pl.when(j > 0)
      def _():
        pltpu.make_async_copy(rows0, os(a), osem0).wait()

      pltpu.async_copy(emb_v.at[idx0], rows0, gsem0)

      pltpu.make_async_copy(xs(b), idx1, isem1).wait()

      @pl.when(j > 0)
      def _():
        pltpu.make_async_copy(rows1, os(b), osem1).wait()

      pltpu.async_copy(emb_v.at[idx1], rows1, gsem1)

      pltpu.make_async_copy(emb_v.at[idx0], rows0, gsem0).wait()
      pltpu.async_copy(rows0, os(a), osem0)

      @pl.when(j < nh - 1)
      def _():
        pltpu.async_copy(xs(a + 2), idx0, isem0)

      pltpu.make_async_copy(emb_v.at[idx1], rows1, gsem1).wait()
      pltpu.async_copy(rows1, os(b), osem1)

      @pl.when(j < nh - 1)
      def _():
        pltpu.async_copy(xs(b + 2), idx1, isem1)

      return carry

    lax.fori_loop(0, nh, step, 0)
    pltpu.make_async_copy(rows0, os(nchunk - 2), osem0).wait()
    pltpu.make_async_copy(rows1, os(nchunk - 1), osem1).wait()

  return k(xf, emb)


def _mlp_body(e_ref, w1_ref, b1_ref, w2_ref, b2_ref, w3_ref, b3_ref, o_ref):
  nct = e_ref.shape[0]
  h = jnp.dot(e_ref[0], w1_ref[0], preferred_element_type=jnp.float32)
  for ct in range(1, nct):
    h = h + jnp.dot(e_ref[ct], w1_ref[ct], preferred_element_type=jnp.float32)
  h = jnp.maximum(h + b1_ref[...], 0.0)
  h = jnp.dot(h, w2_ref[...], preferred_element_type=jnp.float32)
  h = jnp.maximum(h + b2_ref[...], 0.0)
  o = jnp.dot(h, w3_ref[...], preferred_element_type=jnp.float32)
  o_ref[...] = o + b3_ref[...]


def _tc_mlp(e3, W1r, b1, W2, b2, W3, b3, tb):
  nct, bsz, lw = e3.shape
  h1 = W1r.shape[2]
  h2 = W2.shape[1]
  ncls = W3.shape[1]
  grid = (bsz // tb,)
  return pl.pallas_call(
      _mlp_body,
      grid=grid,
      in_specs=[
          pl.BlockSpec((nct, tb, lw), lambda i: (0, i, 0)),
          pl.BlockSpec((nct, lw, h1), lambda i: (0, 0, 0)),
          pl.BlockSpec((1, h1), lambda i: (0, 0)),
          pl.BlockSpec((h1, h2), lambda i: (0, 0)),
          pl.BlockSpec((1, h2), lambda i: (0, 0)),
          pl.BlockSpec((h2, ncls), lambda i: (0, 0)),
          pl.BlockSpec((1, ncls), lambda i: (0, 0)),
      ],
      out_specs=pl.BlockSpec((tb, ncls), lambda i: (i, 0)),
      out_shape=jax.ShapeDtypeStruct((bsz, ncls), jnp.float32),
  )(e3, W1r, b1, W2, b2, W3, b3)


@jax.jit
def kernel(x, emb, W1, b1, W2, b2, W3, b3):
  b, s = x.shape
  v, d = emb.shape
  lw = 128
  g = lw // d
  nct = s // g
  xr = x.reshape(b, nct, g).transpose(1, 0, 2).reshape(-1).astype(jnp.int32)
  rows = _sc_gather(xr, emb, chunk=2048)
  e3 = rows.reshape(nct, b, lw)
  W1r = W1.reshape(nct, lw, W1.shape[1])
  return _tc_mlp(e3, W1r, b1.reshape(1, -1), W2, b2.reshape(1, -1),
                 W3, b3.reshape(1, -1), tb=512)


# speedup vs baseline: 46.2337x; 1.3909x over previous
import functools

import jax
import jax.numpy as jnp
from jax import lax
from jax.experimental import pallas as pl
from jax.experimental.pallas import tpu as pltpu
from jax.experimental.pallas import tpu_sc as plsc

NC = 2
NS = 16
NW = NC * NS


def _sc_gather(x, emb, nb):
  bsz, seq = x.shape
  d = emb.shape[1]
  g = 128 // d
  nct = seq // g
  n = bsz * seq
  chunk = nb * g
  bpw = (bsz // nb) // NW
  npair = (nct + 1) // 2
  pairs_w = bpw * npair
  nhp = pairs_w // 2

  mesh = plsc.VectorSubcoreMesh(core_axis_name="c", subcore_axis_name="s")

  @functools.partial(
      pl.kernel,
      mesh=mesh,
      out_type=jax.ShapeDtypeStruct((n, d), jnp.float32),
      compiler_params=pltpu.CompilerParams(use_tc_tiling_on_sc=False),
      scratch_types=[
          pltpu.VMEM((nb, 2 * g), jnp.int32),
          pltpu.VMEM((nb, 2 * g), jnp.int32),
          pltpu.VMEM((chunk + 8,), jnp.int32),
          pltpu.VMEM((chunk + 16,), jnp.int32),
          pltpu.VMEM((chunk + 8,), jnp.int32),
          pltpu.VMEM((chunk + 16,), jnp.int32),
          pltpu.VMEM((chunk, d), jnp.float32),
          pltpu.VMEM((chunk, d), jnp.float32),
          pltpu.VMEM((chunk, d), jnp.float32),
          pltpu.VMEM((chunk, d), jnp.float32),
          pltpu.VMEM_SHARED(emb.shape, jnp.float32),
          pltpu.SemaphoreType.DMA,
          pltpu.SemaphoreType.DMA,
          pltpu.SemaphoreType.DMA,
          pltpu.SemaphoreType.DMA,
          pltpu.SemaphoreType.DMA,
          pltpu.SemaphoreType.DMA,
          pltpu.SemaphoreType.DMA,
          pltpu.SemaphoreType.DMA,
          pltpu.SemaphoreType.DMA,
          pltpu.SemaphoreType.DMA,
      ],
  )
  def k(x_hbm, emb_hbm, out_hbm,
        xb0, xb1, il0, ih0, il1, ih1, rl0, rh0, rl1, rh1, emb_v,
        xsem0, xsem1, gl0, gh0, gl1, gh1, ol0, oh0, ol1, oh1):
    wid = lax.axis_index("s") * NC + lax.axis_index("c")
    @pl.when(lax.axis_index("s") == 0)
    def _():
      pltpu.sync_copy(emb_hbm, emb_v)

    plsc.subcore_barrier()


    def coords(j):
      bb = j // npair
      p = j % npair
      b0 = (wid * bpw + bb) * nb
      p0 = jnp.minimum(p * 2 * g, seq - 2 * g)
      return b0, p0

    def xs(j, buf):
      b0, p0 = coords(j)
      return x_hbm.at[pl.ds(b0, nb), pl.ds(p0, 2 * g)]

    def outref(j, hi):
      b0, p0 = coords(j)
      ct = p0 // g + hi
      return out_hbm.at[pl.ds(ct * (bsz * g) + b0 * g, chunk)]

    def build(xb, il, ih):
      def body_lo(q, carry):
        il[pl.ds(q * g, 2 * g)] = xb[q]
        return carry
      lax.fori_loop(0, nb, body_lo, 0)

      def body_hi(i, carry):
        q = nb - 1 - i
        ih[pl.ds(q * g, 2 * g)] = xb[q]
        return carry
      lax.fori_loop(0, nb, body_hi, 0)

    def il_sl(il):
      return il.at[pl.ds(0, chunk)]

    def ih_sl(ih):
      return ih.at[pl.ds(g, chunk)]

    pltpu.async_copy(xs(0, 0), xb0, xsem0)
    pltpu.async_copy(xs(1, 1), xb1, xsem1)

    def halfstep(j, xb, il, ih, rl, rh, xsem, gl, gh, ol, oh, jj):
      pltpu.make_async_copy(xs(j, 0), xb, xsem).wait()
      build(xb, il, ih)

      @pl.when(jj < nhp - 1)
      def _():
        pltpu.async_copy(xs(j + 2, 0), xb, xsem)

      @pl.when(jj > 0)
      def _():
        pltpu.make_async_copy(rl, outref(j, 0), ol).wait()
        pltpu.make_async_copy(rh, outref(j, 1), oh).wait()

      pltpu.async_copy(emb_v.at[il_sl(il)], rl, gl)
      pltpu.async_copy(emb_v.at[ih_sl(ih)], rh, gh)

      pltpu.make_async_copy(emb_v.at[il_sl(il)], rl, gl).wait()
      pltpu.async_copy(rl, outref(j, 0), ol)
      pltpu.make_async_copy(emb_v.at[ih_sl(ih)], rh, gh).wait()
      pltpu.async_copy(rh, outref(j, 1), oh)

    def step(jj, carry):
      a = 2 * jj
      halfstep(a, xb0, il0, ih0, rl0, rh0, xsem0, gl0, gh0, ol0, oh0, jj)
      halfstep(a + 1, xb1, il1, ih1, rl1, rh1, xsem1, gl1, gh1, ol1, oh1, jj)
      return carry

    lax.fori_loop(0, nhp, step, 0)
    pltpu.make_async_copy(rl0, outref(pairs_w - 2, 0), ol0).wait()
    pltpu.make_async_copy(rh0, outref(pairs_w - 2, 1), oh0).wait()
    pltpu.make_async_copy(rl1, outref(pairs_w - 1, 0), ol1).wait()
    pltpu.make_async_copy(rh1, outref(pairs_w - 1, 1), oh1).wait()

  return k(x, emb)


def _mlp_body(e_ref, w1_ref, b1_ref, w2_ref, b2_ref, w3_ref, b3_ref, o_ref):
  nct = e_ref.shape[0]
  h = jnp.dot(e_ref[0], w1_ref[0], preferred_element_type=jnp.float32)
  for ct in range(1, nct):
    h = h + jnp.dot(e_ref[ct], w1_ref[ct], preferred_element_type=jnp.float32)
  h = jnp.maximum(h + b1_ref[...], 0.0)
  h = jnp.dot(h, w2_ref[...], preferred_element_type=jnp.float32)
  h = jnp.maximum(h + b2_ref[...], 0.0)
  o = jnp.dot(h, w3_ref[...], preferred_element_type=jnp.float32)
  o_ref[...] = o + b3_ref[...]


def _tc_mlp(e3, W1r, b1, W2, b2, W3, b3, tb):
  nct, bsz, lw = e3.shape
  h1 = W1r.shape[2]
  h2 = W2.shape[1]
  ncls = W3.shape[1]
  grid = (bsz // tb,)
  return pl.pallas_call(
      _mlp_body,
      grid=grid,
      in_specs=[
          pl.BlockSpec((nct, tb, lw), lambda i: (0, i, 0)),
          pl.BlockSpec((nct, lw, h1), lambda i: (0, 0, 0)),
          pl.BlockSpec((1, h1), lambda i: (0, 0)),
          pl.BlockSpec((h1, h2), lambda i: (0, 0)),
          pl.BlockSpec((1, h2), lambda i: (0, 0)),
          pl.BlockSpec((h2, ncls), lambda i: (0, 0)),
          pl.BlockSpec((1, ncls), lambda i: (0, 0)),
      ],
      out_specs=pl.BlockSpec((tb, ncls), lambda i: (i, 0)),
      out_shape=jax.ShapeDtypeStruct((bsz, ncls), jnp.float32),
  )(e3, W1r, b1, W2, b2, W3, b3)


@jax.jit
def kernel(x, emb, W1, b1, W2, b2, W3, b3):
  b, s = x.shape
  v, d = emb.shape
  lw = 128
  nct = s * d // lw
  rows = _sc_gather(x.astype(jnp.int32), emb, nb=128)
  e3 = rows.reshape(nct, b, lw)
  W1r = W1.reshape(nct, lw, W1.shape[1])
  return _tc_mlp(e3, W1r, b1.reshape(1, -1), W2, b2.reshape(1, -1),
                 W3, b3.reshape(1, -1), tb=512)
